# Initial kernel scaffold; baseline (speedup 1.0000x reference)
import jax
import jax.numpy as jnp
from jax.experimental import pallas as pl


def kernel(x, tables):
    raise NotImplementedError("write your pallas kernel here")


# speedup vs baseline: 156.7287x; 156.7287x over previous
import jax
import jax.numpy as jnp
import numpy as np
from jax import lax
from jax.experimental import pallas as pl
from jax.experimental.pallas import tpu as pltpu
from jax.experimental.pallas import tpu_sc as plsc

L = 16
T = 2 ** 14
F = 2
N_MIN = 16.0
N_MAX = 512.0
B_GROWTH = float(np.exp((np.log(N_MAX) - np.log(N_MIN)) / (L - 1)))
NL = [float(np.floor(N_MIN * (B_GROWTH ** i))) for i in range(L)]

P1 = np.int32(np.uint32(2654435761).view(np.int32))
P2 = np.int32(805459861)
MASK = T - 1

N_POINTS = 262144
C = 4096
C16 = C // 16
NCORES = 2
NSUB = 16
NCH = (N_POINTS // NCORES) // C


def _body(x_hbm, tab0_hbm, tab1_hbm, out_hbm,
          x_v, tab0_v, tab1_v, fpair_v, gath_v, out_v, spbuf):
    cid = lax.axis_index("c")
    sid = lax.axis_index("s")
    half_base = cid * (N_POINTS // NCORES)

    pltpu.sync_copy(tab0_hbm.at[sid], tab0_v)
    pltpu.sync_copy(tab1_hbm.at[sid], tab1_v)

    nl_vec = jnp.float32(NL[0])
    for _i in range(1, L):
        nl_vec = jnp.where(sid == _i, jnp.float32(NL[_i]), nl_vec)

    iota = lax.iota(jnp.int32, 16)
    iota3 = iota * 3
    iota2 = iota * 2
    rbase = lax.shift_right_logical(iota, 1)
    cbase = jnp.bitwise_and(iota, 1)

    @pl.loop(0, NCH)
    def _chunk(k):
        row0 = half_base + k * C
        pltpu.sync_copy(x_hbm.at[pl.ds(row0 * 3, C * 3)], x_v)

        @pl.loop(0, C // 16)
        def _grp(g):
            fx = iota3 + g * 48
            px = plsc.load_gather(x_v, [fx])
            py = plsc.load_gather(x_v, [fx + 1])
            pz = plsc.load_gather(x_v, [fx + 2])

            tx = px * nl_vec
            ty = py * nl_vec
            tz = pz * nl_vec
            gx = tx.astype(jnp.int32)
            gy = ty.astype(jnp.int32)
            gz = tz.astype(jnp.int32)
            wx = tx - gx.astype(jnp.float32)
            wy = ty - gy.astype(jnp.float32)
            wz = tz - gz.astype(jnp.float32)

            hy0 = gy * P1
            hy1 = hy0 + P1
            hz0 = gz * P2
            hz1 = hz0 + P2
            a0 = gx & MASK
            a1 = (gx + 1) & MASK
            b = [(hy0 ^ hz0) & MASK, (hy0 ^ hz1) & MASK,
                 (hy1 ^ hz0) & MASK, (hy1 ^ hz1) & MASK]

            res = []
            for tab in (tab0_v, tab1_v):
                yvals = []
                for jj in (0, 1):
                    zvals = []
                    for kk in (0, 1):
                        f0 = plsc.load_gather(tab, [a0 ^ b[2 * jj + kk]])
                        f1 = plsc.load_gather(tab, [a1 ^ b[2 * jj + kk]])
                        zvals.append(f0 + wx * (f1 - f0))
                    yvals.append(zvals[0] + wz * (zvals[1] - zvals[0]))
                res.append(yvals[0] + wy * (yvals[1] - yvals[0]))

            si = iota2 + g * 32
            plsc.store_scatter(fpair_v, [si], res[0])
            plsc.store_scatter(fpair_v, [si + 1], res[1])

        pltpu.sync_copy(fpair_v, spbuf.at[sid])
        plsc.subcore_barrier()
        @pl.loop(0, L)
        def _pull(lv):
            pltpu.sync_copy(spbuf.at[lv, pl.ds(sid * (C16 * 2), C16 * 2)],
                            gath_v.at[lv])
        plsc.subcore_barrier()

        @pl.loop(0, L)
        def _ilv(lv):
            @pl.loop(0, (C16 * 2) // 16)
            def _blk(g):
                v = gath_v[lv, pl.ds(g * 16, 16)]
                ridx = rbase + g * 8
                cidx = cbase + lv * 2
                plsc.store_scatter(out_v, [ridx, cidx], v)

        pltpu.sync_copy(out_v, out_hbm.at[pl.ds(row0 + sid * C16, C16), :])


@jax.jit
def kernel(x, tables):
    n = x.shape[0]
    mesh = plsc.VectorSubcoreMesh(core_axis_name="c", subcore_axis_name="s",
                                  num_cores=NCORES, num_subcores=NSUB)
    run = pl.kernel(
        _body,
        out_type=jax.ShapeDtypeStruct((n, L * F), jnp.float32),
        mesh=mesh,
        compiler_params=pltpu.CompilerParams(needs_layout_passes=False),
        scratch_types=[
            pltpu.VMEM((3 * C,), jnp.float32),
            pltpu.VMEM((T,), jnp.float32),
            pltpu.VMEM((T,), jnp.float32),
            pltpu.VMEM((2 * C,), jnp.float32),
            pltpu.VMEM((L, C16 * 2), jnp.float32),
            pltpu.VMEM((C16, L * F), jnp.float32),
            pltpu.VMEM_SHARED((L, 2 * C), jnp.float32),
        ],
    )
    return run(x.reshape(-1), tables[:, :, 0], tables[:, :, 1])


# speedup vs baseline: 159.5228x; 1.0178x over previous
import jax
import jax.numpy as jnp
import numpy as np
from jax import lax
from jax.experimental import pallas as pl
from jax.experimental.pallas import tpu as pltpu
from jax.experimental.pallas import tpu_sc as plsc

L = 16
T = 2 ** 14
F = 2
N_MIN = 16.0
N_MAX = 512.0
B_GROWTH = float(np.exp((np.log(N_MAX) - np.log(N_MIN)) / (L - 1)))
NL = [float(np.floor(N_MIN * (B_GROWTH ** i))) for i in range(L)]

P1 = np.int32(np.uint32(2654435761).view(np.int32))
P2 = np.int32(805459861)
MASK = T - 1

N_POINTS = 262144
C = 4096
C16 = C // 16
NCORES = 2
NSUB = 16
NCH = (N_POINTS // NCORES) // C


def _body(x_hbm, tab0_hbm, tab1_hbm, out_hbm,
          x_v, tab0_v, tab1_v, fpair_v, gath_v, out_v, spbuf):
    cid = lax.axis_index("c")
    sid = lax.axis_index("s")
    half_base = cid * (N_POINTS // NCORES)

    pltpu.sync_copy(tab0_hbm.at[sid], tab0_v)
    pltpu.sync_copy(tab1_hbm.at[sid], tab1_v)

    nl_vec = jnp.float32(NL[0])
    for _i in range(1, L):
        nl_vec = jnp.where(sid == _i, jnp.float32(NL[_i]), nl_vec)

    iota = lax.iota(jnp.int32, 16)
    iota3 = iota * 3
    iota2 = iota * 2
    rbase = lax.shift_right_logical(iota, 1)
    cbase = jnp.bitwise_and(iota, 1)

    @pl.loop(0, NCH)
    def _chunk(k):
        row0 = half_base + k * C
        pltpu.sync_copy(x_hbm.at[pl.ds(row0 * 3, C * 3)], x_v)

        @pl.loop(0, C // 16, unroll=4)
        def _grp(g):
            fx = iota3 + g * 48
            px = plsc.load_gather(x_v, [fx])
            py = plsc.load_gather(x_v, [fx + 1])
            pz = plsc.load_gather(x_v, [fx + 2])

            tx = px * nl_vec
            ty = py * nl_vec
            tz = pz * nl_vec
            gx = tx.astype(jnp.int32)
            gy = ty.astype(jnp.int32)
            gz = tz.astype(jnp.int32)
            wx = tx - gx.astype(jnp.float32)
            wy = ty - gy.astype(jnp.float32)
            wz = tz - gz.astype(jnp.float32)

            hy0 = gy * P1
            hy1 = hy0 + P1
            hz0 = gz * P2
            hz1 = hz0 + P2
            a0 = gx & MASK
            a1 = (gx + 1) & MASK
            b = [(hy0 ^ hz0) & MASK, (hy0 ^ hz1) & MASK,
                 (hy1 ^ hz0) & MASK, (hy1 ^ hz1) & MASK]

            res = []
            for tab in (tab0_v, tab1_v):
                yvals = []
                for jj in (0, 1):
                    zvals = []
                    for kk in (0, 1):
                        f0 = plsc.load_gather(tab, [a0 ^ b[2 * jj + kk]])
                        f1 = plsc.load_gather(tab, [a1 ^ b[2 * jj + kk]])
                        zvals.append(f0 + wx * (f1 - f0))
                    yvals.append(zvals[0] + wz * (zvals[1] - zvals[0]))
                res.append(yvals[0] + wy * (yvals[1] - yvals[0]))

            si = iota2 + g * 32
            plsc.store_scatter(fpair_v, [si], res[0])
            plsc.store_scatter(fpair_v, [si + 1], res[1])

        pltpu.sync_copy(fpair_v, spbuf.at[sid])
        plsc.subcore_barrier()
        @pl.loop(0, L)
        def _pull(lv):
            pltpu.sync_copy(spbuf.at[lv, pl.ds(sid * (C16 * 2), C16 * 2)],
                            gath_v.at[lv])
        plsc.subcore_barrier()

        @pl.loop(0, L)
        def _ilv(lv):
            @pl.loop(0, (C16 * 2) // 16)
            def _blk(g):
                v = gath_v[lv, pl.ds(g * 16, 16)]
                ridx = rbase + g * 8
                cidx = cbase + lv * 2
                plsc.store_scatter(out_v, [ridx, cidx], v)

        pltpu.sync_copy(out_v, out_hbm.at[pl.ds(row0 + sid * C16, C16), :])


@jax.jit
def kernel(x, tables):
    n = x.shape[0]
    mesh = plsc.VectorSubcoreMesh(core_axis_name="c", subcore_axis_name="s",
                                  num_cores=NCORES, num_subcores=NSUB)
    run = pl.kernel(
        _body,
        out_type=jax.ShapeDtypeStruct((n, L * F), jnp.float32),
        mesh=mesh,
        compiler_params=pltpu.CompilerParams(needs_layout_passes=False),
        scratch_types=[
            pltpu.VMEM((3 * C,), jnp.float32),
            pltpu.VMEM((T,), jnp.float32),
            pltpu.VMEM((T,), jnp.float32),
            pltpu.VMEM((2 * C,), jnp.float32),
            pltpu.VMEM((L, C16 * 2), jnp.float32),
            pltpu.VMEM((C16, L * F), jnp.float32),
            pltpu.VMEM_SHARED((L, 2 * C), jnp.float32),
        ],
    )
    return run(x.reshape(-1), tables[:, :, 0], tables[:, :, 1])


# speedup vs baseline: 187.4242x; 1.1749x over previous
import jax
import jax.numpy as jnp
import numpy as np
from jax import lax
from jax.experimental import pallas as pl
from jax.experimental.pallas import tpu as pltpu
from jax.experimental.pallas import tpu_sc as plsc

L = 16
T = 2 ** 14
F = 2
N_MIN = 16.0
N_MAX = 512.0
B_GROWTH = float(np.exp((np.log(N_MAX) - np.log(N_MIN)) / (L - 1)))
NL = [float(np.floor(N_MIN * (B_GROWTH ** i))) for i in range(L)]

P1 = np.int32(np.uint32(2654435761).view(np.int32))
P2 = np.int32(805459861)
MASK = T - 1

N_POINTS = 262144
C = 4096
C16 = C // 16
NCORES = 2
NSUB = 16
NCH = (N_POINTS // NCORES) // C


def _body(x_hbm, tab0_hbm, tab1_hbm, out_hbm,
          x_v, tab0_v, tab1_v, fpair_v, gath_v, out_v, spbuf):
    cid = lax.axis_index("c")
    sid = lax.axis_index("s")
    half_base = cid * (N_POINTS // NCORES)

    pltpu.sync_copy(tab0_hbm.at[sid], tab0_v)
    pltpu.sync_copy(tab1_hbm.at[sid], tab1_v)

    nl_vec = jnp.float32(NL[0])
    for _i in range(1, L):
        nl_vec = jnp.where(sid == _i, jnp.float32(NL[_i]), nl_vec)

    iota = lax.iota(jnp.int32, 16)
    iota3 = iota * 3
    iota2 = iota * 2
    rbase = lax.shift_right_logical(iota, 1)
    cbase = jnp.bitwise_and(iota, 1)

    @pl.loop(0, NCH)
    def _chunk(k):
        row0 = half_base + k * C
        pltpu.sync_copy(x_hbm.at[pl.ds(row0 * 3, C * 3)], x_v)

        @plsc.parallel_loop(0, C // 16, unroll=4)
        def _grp(g):
            fx = iota3 + g * 48
            px = plsc.load_gather(x_v, [fx])
            py = plsc.load_gather(x_v, [fx + 1])
            pz = plsc.load_gather(x_v, [fx + 2])

            tx = px * nl_vec
            ty = py * nl_vec
            tz = pz * nl_vec
            gx = tx.astype(jnp.int32)
            gy = ty.astype(jnp.int32)
            gz = tz.astype(jnp.int32)
            wx = tx - gx.astype(jnp.float32)
            wy = ty - gy.astype(jnp.float32)
            wz = tz - gz.astype(jnp.float32)

            hy0 = gy * P1
            hy1 = hy0 + P1
            hz0 = gz * P2
            hz1 = hz0 + P2
            a0 = gx & MASK
            a1 = (gx + 1) & MASK
            b = [(hy0 ^ hz0) & MASK, (hy0 ^ hz1) & MASK,
                 (hy1 ^ hz0) & MASK, (hy1 ^ hz1) & MASK]

            res = []
            for tab in (tab0_v, tab1_v):
                yvals = []
                for jj in (0, 1):
                    zvals = []
                    for kk in (0, 1):
                        f0 = plsc.load_gather(tab, [a0 ^ b[2 * jj + kk]])
                        f1 = plsc.load_gather(tab, [a1 ^ b[2 * jj + kk]])
                        zvals.append(f0 + wx * (f1 - f0))
                    yvals.append(zvals[0] + wz * (zvals[1] - zvals[0]))
                res.append(yvals[0] + wy * (yvals[1] - yvals[0]))

            si = iota2 + g * 32
            plsc.store_scatter(fpair_v, [si], res[0])
            plsc.store_scatter(fpair_v, [si + 1], res[1])

        pltpu.sync_copy(fpair_v, spbuf.at[sid])
        plsc.subcore_barrier()
        @pl.loop(0, L)
        def _pull(lv):
            pltpu.sync_copy(spbuf.at[lv, pl.ds(sid * (C16 * 2), C16 * 2)],
                            gath_v.at[lv])
        plsc.subcore_barrier()

        @pl.loop(0, L)
        def _ilv(lv):
            @pl.loop(0, (C16 * 2) // 16)
            def _blk(g):
                v = gath_v[lv, pl.ds(g * 16, 16)]
                ridx = rbase + g * 8
                cidx = cbase + lv * 2
                plsc.store_scatter(out_v, [ridx, cidx], v)

        pltpu.sync_copy(out_v, out_hbm.at[pl.ds(row0 + sid * C16, C16), :])


@jax.jit
def kernel(x, tables):
    n = x.shape[0]
    mesh = plsc.VectorSubcoreMesh(core_axis_name="c", subcore_axis_name="s",
                                  num_cores=NCORES, num_subcores=NSUB)
    run = pl.kernel(
        _body,
        out_type=jax.ShapeDtypeStruct((n, L * F), jnp.float32),
        mesh=mesh,
        compiler_params=pltpu.CompilerParams(needs_layout_passes=False),
        scratch_types=[
            pltpu.VMEM((3 * C,), jnp.float32),
            pltpu.VMEM((T,), jnp.float32),
            pltpu.VMEM((T,), jnp.float32),
            pltpu.VMEM((2 * C,), jnp.float32),
            pltpu.VMEM((L, C16 * 2), jnp.float32),
            pltpu.VMEM((C16, L * F), jnp.float32),
            pltpu.VMEM_SHARED((L, 2 * C), jnp.float32),
        ],
    )
    return run(x.reshape(-1), tables[:, :, 0], tables[:, :, 1])
